# Initial kernel scaffold; baseline (speedup 1.0000x reference)
import functools

import jax
import jax.numpy as jnp
from jax import lax
from jax.experimental import pallas as pl
from jax.experimental.pallas import tpu as pltpu
from jax.experimental.pallas import tpu_sc as plsc

N = 50000
D = 256
B = 512
NITERS = 6
L = 16
KV = D // L
NC = 2
NS = 16
NW = NC * NS
SPW = B // NW
CH = 128
NEG = -1e30


def _attn_body(x_hbm, offs_hbm, q_hbm, r_hbm, offs_v, qbuf, xbuf, rbuf):
    cid = lax.axis_index("c")
    sid = lax.axis_index("s")
    wid = sid * NC + cid
    seg0 = wid * SPW
    pltpu.sync_copy(offs_hbm.at[pl.ds(seg0, 24)], offs_v)
    pltpu.sync_copy(q_hbm.at[pl.ds(seg0, SPW)], qbuf)

    for j in range(SPW):
        lo = offs_v[j]
        hi = offs_v[j + 1]
        qv = [qbuf[j, pl.ds(k * L, L)] for k in range(KV)]
        nch = (hi - lo + (CH - 1)) // CH

        def chunk_body(ci, carry, lo=lo, hi=hi, qv=qv):
            pos = lo + ci * CH
            pos_c = jnp.minimum(pos, N - CH)
            off = pos - pos_c
            pltpu.sync_copy(x_hbm.at[pl.ds(pos_c, CH)], xbuf)
            nrows = jnp.minimum(hi - pos, CH)

            def row_body(i, rc, off=off, qv=qv):
                m, dacc = rc[0], rc[1]
                rho = rc[2:]
                bi = off + i
                xk = [xbuf[bi, pl.ds(k * L, L)] for k in range(KV)]
                acc = xk[0] * qv[0]
                for k in range(1, KV):
                    acc = acc + xk[k] * qv[k]
                e = jnp.broadcast_to(jnp.sum(acc), (L,))
                m_new = jnp.maximum(m, e)
                sc = jnp.exp(m - m_new)
                w = jnp.exp(e - m_new)
                dacc = dacc * sc + w
                rho = tuple(rho[k] * sc + w * xk[k] for k in range(KV))
                return (m_new, dacc) + rho

            return lax.fori_loop(0, nrows, row_body, carry)

        zero = jnp.zeros((L,), jnp.float32)
        init = (jnp.full((L,), NEG, jnp.float32), zero) + (zero,) * KV
        res = lax.fori_loop(0, nch, chunk_body, init)
        dacc = res[1]
        inv = jnp.where(dacc > 0.0, 1.0 / dacc, 0.0)
        for k in range(KV):
            rbuf[j, pl.ds(k * L, L)] = res[2 + k] * inv

    pltpu.sync_copy(rbuf, r_hbm.at[pl.ds(seg0, SPW)])


_attn = functools.partial(
    pl.kernel,
    mesh=plsc.VectorSubcoreMesh(core_axis_name="c", subcore_axis_name="s"),
    out_type=jax.ShapeDtypeStruct((B, D), jnp.float32),
    scratch_types=[
        pltpu.VMEM((24,), jnp.int32),
        pltpu.VMEM((SPW, D), jnp.float32),
        pltpu.VMEM((CH, D), jnp.float32),
        pltpu.VMEM((SPW, D), jnp.float32),
    ],
)(_attn_body)


def _lstm_body(qp_ref, rp_ref, h_ref, c_ref,
               wih0q_ref, wih0r_ref, whh0_ref, b0_ref,
               wih1_ref, whh1_ref, b1_ref,
               wih2_ref, whh2_ref, b2_ref,
               q_out, h_out, c_out):
    def dot(a, b):
        return lax.dot_general(a, b, (((1,), (0,)), ((), ())),
                               preferred_element_type=jnp.float32)

    def cell(gates, c_prev):
        i = jax.nn.sigmoid(gates[:, 0:D])
        f = jax.nn.sigmoid(gates[:, D:2 * D])
        g = jnp.tanh(gates[:, 2 * D:3 * D])
        o = jax.nn.sigmoid(gates[:, 3 * D:4 * D])
        c_new = f * c_prev + i * g
        h_new = o * jnp.tanh(c_new)
        return h_new, c_new

    g0 = (dot(qp_ref[...], wih0q_ref[...]) + dot(rp_ref[...], wih0r_ref[...])
          + dot(h_ref[0], whh0_ref[...]) + b0_ref[...])
    h0, c0 = cell(g0, c_ref[0])
    g1 = dot(h0, wih1_ref[...]) + dot(h_ref[1], whh1_ref[...]) + b1_ref[...]
    h1, c1 = cell(g1, c_ref[1])
    g2 = dot(h1, wih2_ref[...]) + dot(h_ref[2], whh2_ref[...]) + b2_ref[...]
    h2, c2 = cell(g2, c_ref[2])
    h_out[0], h_out[1], h_out[2] = h0, h1, h2
    c_out[0], c_out[1], c_out[2] = c0, c1, c2
    q_out[...] = h2


_lstm = pl.pallas_call(
    _lstm_body,
    out_shape=(
        jax.ShapeDtypeStruct((B, D), jnp.float32),
        jax.ShapeDtypeStruct((3, B, D), jnp.float32),
        jax.ShapeDtypeStruct((3, B, D), jnp.float32),
    ),
)


def kernel(x, batch, W_ih_0, W_hh_0, b_ih_0, b_hh_0,
           W_ih_1, W_hh_1, b_ih_1, b_hh_1,
           W_ih_2, W_hh_2, b_ih_2, b_hh_2):
    batch32 = batch.astype(jnp.int32)
    offs = jnp.searchsorted(batch32, jnp.arange(B + 1, dtype=jnp.int32),
                            side="left").astype(jnp.int32)
    offs_pad = jnp.concatenate([offs, jnp.full((15,), N, jnp.int32)])

    wih0q = W_ih_0[:, :D].T
    wih0r = W_ih_0[:, D:].T
    whh0 = W_hh_0.T
    wih1, whh1 = W_ih_1.T, W_hh_1.T
    wih2, whh2 = W_ih_2.T, W_hh_2.T
    b0 = (b_ih_0 + b_hh_0).reshape(1, 4 * D)
    b1 = (b_ih_1 + b_hh_1).reshape(1, 4 * D)
    b2 = (b_ih_2 + b_hh_2).reshape(1, 4 * D)

    qp = jnp.zeros((B, D), jnp.float32)
    rp = jnp.zeros((B, D), jnp.float32)
    h = jnp.zeros((3, B, D), jnp.float32)
    c = jnp.zeros((3, B, D), jnp.float32)
    for _ in range(NITERS):
        q, h, c = _lstm(qp, rp, h, c, wih0q, wih0r, whh0, b0,
                        wih1, whh1, b1, wih2, whh2, b2)
        r = _attn(x, offs_pad, q)
        qp, rp = q, r
    return jnp.concatenate([qp, rp], axis=1)


# speedup vs baseline: 9.2480x; 9.2480x over previous
import functools

import jax
import jax.numpy as jnp
from jax import lax
from jax.experimental import pallas as pl
from jax.experimental.pallas import tpu as pltpu
from jax.experimental.pallas import tpu_sc as plsc

N = 50000
D = 256
B = 512
NITERS = 6
L = 16
KV = D // L
NC = 2
NS = 16
NW = NC * NS
SPW = B // NW
CH = 128
NEG = -1e30


def _attn_body(x_hbm, offs_hbm, q_hbm, r_hbm, offs_v, qbuf, xbuf, rbuf):
    cid = lax.axis_index("c")
    sid = lax.axis_index("s")
    wid = sid * NC + cid
    seg0 = wid * SPW
    pltpu.sync_copy(offs_hbm.at[pl.ds(seg0, 24)], offs_v)
    pltpu.sync_copy(q_hbm.at[pl.ds(seg0, SPW)], qbuf)

    o_lo = offs_v[pl.ds(0, 16)]
    o_hi = offs_v[pl.ds(8, 16)]

    for j in range(SPW):
        lo = o_lo[j]
        hi = o_lo[j + 1] if j < 15 else o_hi[8]
        qv = [qbuf[j, pl.ds(k * L, L)] for k in range(KV)]

        lo8 = (lo // 8) * 8
        nch = (hi - lo8 + (CH - 1)) // CH

        def chunk_body(ci, carry, lo=lo, hi=hi, lo8=lo8, qv=qv):
            pos = lo8 + ci * CH
            pos_c = jnp.minimum(pos, N - CH)
            pltpu.sync_copy(x_hbm.at[pl.ds(pos_c, CH)], xbuf)
            start_loc = jnp.maximum(lo, pos) - pos_c
            end_loc = jnp.minimum(hi, pos + CH) - pos_c

            def row_body(bi, rc, qv=qv):
                m, dacc = rc[0], rc[1]
                rho = rc[2:]
                xk = [xbuf[bi, pl.ds(k * L, L)] for k in range(KV)]
                acc = xk[0] * qv[0]
                for k in range(1, KV):
                    acc = acc + xk[k] * qv[k]
                e = jnp.broadcast_to(jnp.sum(acc), (L,))
                m_new = jnp.maximum(m, e)
                sc = jnp.exp(m - m_new)
                w = jnp.exp(e - m_new)
                dacc = dacc * sc + w
                rho = tuple(rho[k] * sc + w * xk[k] for k in range(KV))
                return (m_new, dacc) + rho

            return lax.fori_loop(start_loc, end_loc, row_body, carry)

        zero = jnp.zeros((L,), jnp.float32)
        init = (jnp.full((L,), NEG, jnp.float32), zero) + (zero,) * KV
        res = lax.fori_loop(0, nch, chunk_body, init)
        dacc = res[1]
        inv = jnp.where(dacc > 0.0, 1.0 / dacc, 0.0)
        for k in range(KV):
            rbuf[j, pl.ds(k * L, L)] = res[2 + k] * inv

    pltpu.sync_copy(rbuf, r_hbm.at[pl.ds(seg0, SPW)])


_attn = functools.partial(
    pl.kernel,
    mesh=plsc.VectorSubcoreMesh(core_axis_name="c", subcore_axis_name="s"),
    out_type=jax.ShapeDtypeStruct((B, D), jnp.float32),
    compiler_params=pltpu.CompilerParams(needs_layout_passes=False),
    scratch_types=[
        pltpu.VMEM((24,), jnp.int32),
        pltpu.VMEM((SPW, D), jnp.float32),
        pltpu.VMEM((CH, D), jnp.float32),
        pltpu.VMEM((SPW, D), jnp.float32),
    ],
)(_attn_body)


def _lstm_body(qp_ref, rp_ref, h_ref, c_ref,
               wih0q_ref, wih0r_ref, whh0_ref, b0_ref,
               wih1_ref, whh1_ref, b1_ref,
               wih2_ref, whh2_ref, b2_ref,
               q_out, h_out, c_out):
    def dot(a, b):
        return lax.dot_general(a, b, (((1,), (0,)), ((), ())),
                               preferred_element_type=jnp.float32)

    def cell(gates, c_prev):
        i = jax.nn.sigmoid(gates[:, 0:D])
        f = jax.nn.sigmoid(gates[:, D:2 * D])
        g = jnp.tanh(gates[:, 2 * D:3 * D])
        o = jax.nn.sigmoid(gates[:, 3 * D:4 * D])
        c_new = f * c_prev + i * g
        h_new = o * jnp.tanh(c_new)
        return h_new, c_new

    g0 = (dot(qp_ref[...], wih0q_ref[...]) + dot(rp_ref[...], wih0r_ref[...])
          + dot(h_ref[0], whh0_ref[...]) + b0_ref[...])
    h0, c0 = cell(g0, c_ref[0])
    g1 = dot(h0, wih1_ref[...]) + dot(h_ref[1], whh1_ref[...]) + b1_ref[...]
    h1, c1 = cell(g1, c_ref[1])
    g2 = dot(h1, wih2_ref[...]) + dot(h_ref[2], whh2_ref[...]) + b2_ref[...]
    h2, c2 = cell(g2, c_ref[2])
    h_out[0], h_out[1], h_out[2] = h0, h1, h2
    c_out[0], c_out[1], c_out[2] = c0, c1, c2
    q_out[...] = h2


_lstm = pl.pallas_call(
    _lstm_body,
    out_shape=(
        jax.ShapeDtypeStruct((B, D), jnp.float32),
        jax.ShapeDtypeStruct((3, B, D), jnp.float32),
        jax.ShapeDtypeStruct((3, B, D), jnp.float32),
    ),
)


def kernel(x, batch, W_ih_0, W_hh_0, b_ih_0, b_hh_0,
           W_ih_1, W_hh_1, b_ih_1, b_hh_1,
           W_ih_2, W_hh_2, b_ih_2, b_hh_2):
    batch32 = batch.astype(jnp.int32)
    offs = jnp.searchsorted(batch32, jnp.arange(B + 1, dtype=jnp.int32),
                            side="left").astype(jnp.int32)
    offs_pad = jnp.concatenate([offs, jnp.full((15,), N, jnp.int32)])

    wih0q = W_ih_0[:, :D].T
    wih0r = W_ih_0[:, D:].T
    whh0 = W_hh_0.T
    wih1, whh1 = W_ih_1.T, W_hh_1.T
    wih2, whh2 = W_ih_2.T, W_hh_2.T
    b0 = (b_ih_0 + b_hh_0).reshape(1, 4 * D)
    b1 = (b_ih_1 + b_hh_1).reshape(1, 4 * D)
    b2 = (b_ih_2 + b_hh_2).reshape(1, 4 * D)

    qp = jnp.zeros((B, D), jnp.float32)
    rp = jnp.zeros((B, D), jnp.float32)
    h = jnp.zeros((3, B, D), jnp.float32)
    c = jnp.zeros((3, B, D), jnp.float32)
    for _ in range(NITERS):
        q, h, c = _lstm(qp, rp, h, c, wih0q, wih0r, whh0, b0,
                        wih1, whh1, b1, wih2, whh2, b2)
        r = _attn(x, offs_pad, q)
        qp, rp = q, r
    return jnp.concatenate([qp, rp], axis=1)
